# Initial kernel scaffold; baseline (speedup 1.0000x reference)
import functools

import jax
import jax.numpy as jnp
from jax import lax
from jax.experimental import pallas as pl
from jax.experimental.pallas import tpu as pltpu
from jax.experimental.pallas import tpu_sc as plsc

N = 10000
D = 128
E = 320000
NC = 2
NS = 16
NW = NC * NS
E_PER_W = E // NW
CHUNK = 80
N_CHUNKS = E_PER_W // CHUNK
ROWS_PER_TILE = N // NS
WB = 125
DEG_W = 16
TC_B = 500
TC_GRID = N // TC_B


def _mesh():
    return plsc.VectorSubcoreMesh(
        core_axis_name="c", subcore_axis_name="s", num_cores=NC, num_subcores=NS
    )


def _zero_f32(ref, rows, width):
    zeros = jnp.zeros((16,), jnp.float32)
    def body(r, _):
        for j in range(width // 16):
            ref[r, pl.ds(j * 16, 16)] = zeros
        return 0
    lax.fori_loop(0, rows, body, 0)


def _sc_degree(dst):
    @functools.partial(
        pl.kernel,
        out_type=jax.ShapeDtypeStruct((NC * N, DEG_W), jnp.float32),
        mesh=_mesh(),
        scratch_types=[
            pltpu.VMEM((CHUNK,), jnp.int32),
            pltpu.VMEM((CHUNK, DEG_W), jnp.float32),
            pltpu.VMEM((WB, DEG_W), jnp.float32),
            pltpu.VMEM_SHARED((N, DEG_W), jnp.float32),
        ],
    )
    def deg_kernel(dst_hbm, out_hbm, idx_v, ones_v, buf_v, acc_sh):
        c = lax.axis_index("c")
        s = lax.axis_index("s")
        wid = c * NS + s

        ones = jnp.full((16,), 1.0, jnp.float32)
        def fill_ones(r, _):
            ones_v[r, pl.ds(0, 16)] = ones
            return 0
        lax.fori_loop(0, CHUNK, fill_ones, 0)
        _zero_f32(buf_v, WB, DEG_W)
        for j in range(ROWS_PER_TILE // WB):
            pltpu.sync_copy(buf_v, acc_sh.at[pl.ds(s * ROWS_PER_TILE + j * WB, WB)])
        plsc.subcore_barrier()

        def body(ch, _):
            base = pl.multiple_of(wid * E_PER_W + ch * CHUNK, 8)
            pltpu.sync_copy(dst_hbm.at[pl.ds(base, CHUNK)], idx_v)
            pltpu.sync_copy(ones_v, acc_sh.at[idx_v], add=True)
            return 0
        lax.fori_loop(0, N_CHUNKS, body, 0)
        plsc.subcore_barrier()

        for j in range(ROWS_PER_TILE // WB):
            row = s * ROWS_PER_TILE + j * WB
            pltpu.sync_copy(acc_sh.at[pl.ds(row, WB)], buf_v)
            pltpu.sync_copy(buf_v, out_hbm.at[pl.ds(c * N + row, WB)])

    return deg_kernel(dst)


def _sc_aggregate(t, src, dst):
    @functools.partial(
        pl.kernel,
        out_type=jax.ShapeDtypeStruct((NC * N, D), jnp.float32),
        mesh=_mesh(),
        scratch_types=[
            pltpu.VMEM((CHUNK,), jnp.int32),
            pltpu.VMEM((CHUNK,), jnp.int32),
            pltpu.VMEM((CHUNK, D), jnp.float32),
            pltpu.VMEM((WB, D), jnp.float32),
            pltpu.VMEM_SHARED((N, D), jnp.float32),
            pltpu.SemaphoreType.DMA,
        ],
    )
    def agg_kernel(t_hbm, src_hbm, dst_hbm, out_hbm,
                   src_v, dst_v, rows_v, buf_v, acc_sh, sem):
        c = lax.axis_index("c")
        s = lax.axis_index("s")
        wid = c * NS + s

        _zero_f32(buf_v, WB, D)
        for j in range(ROWS_PER_TILE // WB):
            pltpu.sync_copy(buf_v, acc_sh.at[pl.ds(s * ROWS_PER_TILE + j * WB, WB)])
        plsc.subcore_barrier()

        def body(ch, _):
            base = pl.multiple_of(wid * E_PER_W + ch * CHUNK, 8)
            pltpu.sync_copy(src_hbm.at[pl.ds(base, CHUNK)], src_v)
            pltpu.sync_copy(dst_hbm.at[pl.ds(base, CHUNK)], dst_v)
            pltpu.async_copy(t_hbm.at[src_v], rows_v, sem).wait()
            pltpu.sync_copy(rows_v, acc_sh.at[dst_v], add=True)
            return 0
        lax.fori_loop(0, N_CHUNKS, body, 0)
        plsc.subcore_barrier()

        for j in range(ROWS_PER_TILE // WB):
            row = s * ROWS_PER_TILE + j * WB
            pltpu.sync_copy(acc_sh.at[pl.ds(row, WB)], buf_v)
            pltpu.sync_copy(buf_v, out_hbm.at[pl.ds(c * N + row, WB)])

    return agg_kernel(t, src, dst)


def _dinv_block(d0_ref, d1_ref):
    deg = d0_ref[:, 0:1] + d1_ref[:, 0:1] + 1.0
    return lax.rsqrt(deg)


def _tc1_body(x_ref, w_ref, d0_ref, d1_ref, o_ref):
    dinv = _dinv_block(d0_ref, d1_ref)
    o_ref[...] = dinv * jnp.dot(x_ref[...], w_ref[...],
                                preferred_element_type=jnp.float32)


def _tc2_body(t_ref, a0_ref, a1_ref, d0_ref, d1_ref, b_ref, w_ref, o_ref):
    dinv = _dinv_block(d0_ref, d1_ref)
    z = dinv * (a0_ref[...] + a1_ref[...] + t_ref[...]) + b_ref[...]
    h = jnp.maximum(z, 0.0)
    o_ref[...] = dinv * jnp.dot(h, w_ref[...], preferred_element_type=jnp.float32)


def _tc3_body(t_ref, a0_ref, a1_ref, d0_ref, d1_ref, b_ref, o_ref):
    dinv = _dinv_block(d0_ref, d1_ref)
    z = dinv * (a0_ref[...] + a1_ref[...] + t_ref[...]) + b_ref[...]
    m = jnp.max(z, axis=1, keepdims=True)
    ez = jnp.exp(z - m)
    o_ref[...] = z - m - jnp.log(jnp.sum(ez, axis=1, keepdims=True))


_row_blk = pl.BlockSpec((TC_B, D), lambda i: (i, 0))
_deg0_blk = pl.BlockSpec((TC_B, DEG_W), lambda i: (i, 0))
_deg1_blk = pl.BlockSpec((TC_B, DEG_W), lambda i: (i + TC_GRID, 0))
_agg0_blk = pl.BlockSpec((TC_B, D), lambda i: (i, 0))
_agg1_blk = pl.BlockSpec((TC_B, D), lambda i: (i + TC_GRID, 0))
_w_blk = pl.BlockSpec((D, D), lambda i: (0, 0))
_b_blk = pl.BlockSpec((1, D), lambda i: (0, 0))
_out_sds = jax.ShapeDtypeStruct((N, D), jnp.float32)


def _tc1(x, W1, deg2):
    return pl.pallas_call(
        _tc1_body,
        grid=(TC_GRID,),
        in_specs=[_row_blk, _w_blk, _deg0_blk, _deg1_blk],
        out_specs=_row_blk,
        out_shape=_out_sds,
    )(x, W1, deg2, deg2)


def _tc2(t1, agg1, deg2, b1, W2):
    return pl.pallas_call(
        _tc2_body,
        grid=(TC_GRID,),
        in_specs=[_row_blk, _agg0_blk, _agg1_blk, _deg0_blk, _deg1_blk,
                  _b_blk, _w_blk],
        out_specs=_row_blk,
        out_shape=_out_sds,
    )(t1, agg1, agg1, deg2, deg2, b1.reshape(1, D), W2)


def _tc3(t2, agg2, deg2, b2):
    return pl.pallas_call(
        _tc3_body,
        grid=(TC_GRID,),
        in_specs=[_row_blk, _agg0_blk, _agg1_blk, _deg0_blk, _deg1_blk, _b_blk],
        out_specs=_row_blk,
        out_shape=_out_sds,
    )(t2, agg2, agg2, deg2, deg2, b2.reshape(1, D))


def kernel(x, edge_index, W1, b1, W2, b2):
    src = edge_index[0].astype(jnp.int32)
    dst = edge_index[1].astype(jnp.int32)
    deg2 = _sc_degree(dst)
    t1 = _tc1(x, W1, deg2)
    agg1 = _sc_aggregate(t1, src, dst)
    t2 = _tc2(t1, agg1, deg2, b1, W2)
    agg2 = _sc_aggregate(t2, src, dst)
    return _tc3(t2, agg2, deg2, b2)


# speedup vs baseline: 13.0821x; 13.0821x over previous
import functools

import jax
import jax.numpy as jnp
from jax import lax
from jax.experimental import pallas as pl
from jax.experimental.pallas import tpu as pltpu
from jax.experimental.pallas import tpu_sc as plsc

N = 10000
D = 128
E = 320000
NC = 2
NS = 16
NW = NC * NS
E_PER_W = E // NW
CHUNK = 80
N_CHUNKS = E_PER_W // CHUNK
WB = 80
N_WBCH = N // WB
WB_PER_TILE = -(-N_WBCH // NS)
DEG_W = 16
TC_B = 1000
TC_GRID = N // TC_B


def _mesh():
    return plsc.VectorSubcoreMesh(
        core_axis_name="c", subcore_axis_name="s", num_cores=NC, num_subcores=NS
    )


def _zero_f32(ref, rows, width):
    zeros = jnp.zeros((16,), jnp.float32)
    def body(r, _):
        for j in range(width // 16):
            ref[r, pl.ds(j * 16, 16)] = zeros
        return 0
    lax.fori_loop(0, rows, body, 0)


def _sc_degree(dst):
    @functools.partial(
        pl.kernel,
        out_type=jax.ShapeDtypeStruct((NC * N, DEG_W), jnp.float32),
        mesh=_mesh(),
        scratch_types=[
            pltpu.VMEM((CHUNK,), jnp.int32),
            pltpu.VMEM((CHUNK, DEG_W), jnp.float32),
            pltpu.VMEM((WB, DEG_W), jnp.float32),
            pltpu.VMEM_SHARED((N, DEG_W), jnp.float32),
        ],
        compiler_params=pltpu.CompilerParams(use_tc_tiling_on_sc=False),
    )
    def deg_kernel(dst_hbm, out_hbm, idx_v, ones_v, buf_v, acc_sh):
        c = lax.axis_index("c")
        s = lax.axis_index("s")
        wid = c * NS + s

        ones = jnp.full((16,), 1.0, jnp.float32)
        def fill_ones(r, _):
            ones_v[r, pl.ds(0, 16)] = ones
            return 0
        lax.fori_loop(0, CHUNK, fill_ones, 0)
        _zero_f32(buf_v, WB, DEG_W)
        for jj in range(WB_PER_TILE):
            j = jj * NS + s
            @pl.when(j < N_WBCH)
            def _():
                pltpu.sync_copy(buf_v, acc_sh.at[pl.ds(j * WB, WB)])
        plsc.subcore_barrier()

        def body(ch, _):
            base = pl.multiple_of(wid * E_PER_W + ch * CHUNK, 8)
            pltpu.sync_copy(dst_hbm.at[pl.ds(base, CHUNK)], idx_v)
            pltpu.sync_copy(ones_v, acc_sh.at[idx_v], add=True)
            return 0
        lax.fori_loop(0, N_CHUNKS, body, 0)
        plsc.subcore_barrier()

        for jj in range(WB_PER_TILE):
            j = jj * NS + s
            @pl.when(j < N_WBCH)
            def _():
                row = pl.multiple_of(j * WB, 8)
                pltpu.sync_copy(acc_sh.at[pl.ds(row, WB)], buf_v)
                pltpu.sync_copy(buf_v, out_hbm.at[pl.ds(c * N + row, WB)])

    return deg_kernel(dst)


def _sc_aggregate(t, src, dst):
    @functools.partial(
        pl.kernel,
        out_type=jax.ShapeDtypeStruct((NC * N, D), jnp.float32),
        mesh=_mesh(),
        scratch_types=[
            pltpu.VMEM((CHUNK,), jnp.int32),
            pltpu.VMEM((CHUNK,), jnp.int32),
            pltpu.VMEM((CHUNK, D), jnp.float32),
            pltpu.VMEM((WB, D), jnp.float32),
            pltpu.VMEM_SHARED((N, D), jnp.float32),
            pltpu.SemaphoreType.DMA,
        ],
    )
    def agg_kernel(t_hbm, src_hbm, dst_hbm, out_hbm,
                   src_v, dst_v, rows_v, buf_v, acc_sh, sem):
        c = lax.axis_index("c")
        s = lax.axis_index("s")
        wid = c * NS + s

        _zero_f32(buf_v, WB, D)
        for jj in range(WB_PER_TILE):
            j = jj * NS + s
            @pl.when(j < N_WBCH)
            def _():
                pltpu.sync_copy(buf_v, acc_sh.at[pl.ds(j * WB, WB)])
        plsc.subcore_barrier()

        def body(ch, _):
            base = pl.multiple_of(wid * E_PER_W + ch * CHUNK, 8)
            pltpu.sync_copy(src_hbm.at[pl.ds(base, CHUNK)], src_v)
            pltpu.sync_copy(dst_hbm.at[pl.ds(base, CHUNK)], dst_v)
            pltpu.async_copy(t_hbm.at[src_v], rows_v, sem).wait()
            pltpu.sync_copy(rows_v, acc_sh.at[dst_v], add=True)
            return 0
        lax.fori_loop(0, N_CHUNKS, body, 0)
        plsc.subcore_barrier()

        for jj in range(WB_PER_TILE):
            j = jj * NS + s
            @pl.when(j < N_WBCH)
            def _():
                row = pl.multiple_of(j * WB, 8)
                pltpu.sync_copy(acc_sh.at[pl.ds(row, WB)], buf_v)
                pltpu.sync_copy(buf_v, out_hbm.at[pl.ds(c * N + row, WB)])

    return agg_kernel(t, src, dst)


def _dinv_block(d0_ref, d1_ref):
    deg = d0_ref[:, 0:1] + d1_ref[:, 0:1] + 1.0
    return lax.rsqrt(deg)


def _tc1_body(x_ref, w_ref, d0_ref, d1_ref, o_ref):
    dinv = _dinv_block(d0_ref, d1_ref)
    o_ref[...] = dinv * jnp.dot(x_ref[...], w_ref[...],
                                preferred_element_type=jnp.float32)


def _tc2_body(t_ref, a0_ref, a1_ref, d0_ref, d1_ref, b_ref, w_ref, o_ref):
    dinv = _dinv_block(d0_ref, d1_ref)
    z = dinv * (a0_ref[...] + a1_ref[...] + t_ref[...]) + b_ref[...]
    h = jnp.maximum(z, 0.0)
    o_ref[...] = dinv * jnp.dot(h, w_ref[...], preferred_element_type=jnp.float32)


def _tc3_body(t_ref, a0_ref, a1_ref, d0_ref, d1_ref, b_ref, o_ref):
    dinv = _dinv_block(d0_ref, d1_ref)
    z = dinv * (a0_ref[...] + a1_ref[...] + t_ref[...]) + b_ref[...]
    m = jnp.max(z, axis=1, keepdims=True)
    ez = jnp.exp(z - m)
    o_ref[...] = z - m - jnp.log(jnp.sum(ez, axis=1, keepdims=True))


_row_blk = pl.BlockSpec((TC_B, D), lambda i: (i, 0))
_deg0_blk = pl.BlockSpec((TC_B, DEG_W), lambda i: (i, 0))
_deg1_blk = pl.BlockSpec((TC_B, DEG_W), lambda i: (i + TC_GRID, 0))
_agg0_blk = pl.BlockSpec((TC_B, D), lambda i: (i, 0))
_agg1_blk = pl.BlockSpec((TC_B, D), lambda i: (i + TC_GRID, 0))
_w_blk = pl.BlockSpec((D, D), lambda i: (0, 0))
_b_blk = pl.BlockSpec((1, D), lambda i: (0, 0))
_out_sds = jax.ShapeDtypeStruct((N, D), jnp.float32)


def _tc1(x, W1, deg2):
    return pl.pallas_call(
        _tc1_body,
        grid=(TC_GRID,),
        in_specs=[_row_blk, _w_blk, _deg0_blk, _deg1_blk],
        out_specs=_row_blk,
        out_shape=_out_sds,
    )(x, W1, deg2, deg2)


def _tc2(t1, agg1, deg2, b1, W2):
    return pl.pallas_call(
        _tc2_body,
        grid=(TC_GRID,),
        in_specs=[_row_blk, _agg0_blk, _agg1_blk, _deg0_blk, _deg1_blk,
                  _b_blk, _w_blk],
        out_specs=_row_blk,
        out_shape=_out_sds,
    )(t1, agg1, agg1, deg2, deg2, b1.reshape(1, D), W2)


def _tc3(t2, agg2, deg2, b2):
    return pl.pallas_call(
        _tc3_body,
        grid=(TC_GRID,),
        in_specs=[_row_blk, _agg0_blk, _agg1_blk, _deg0_blk, _deg1_blk, _b_blk],
        out_specs=_row_blk,
        out_shape=_out_sds,
    )(t2, agg2, agg2, deg2, deg2, b2.reshape(1, D))


def kernel(x, edge_index, W1, b1, W2, b2):
    src = edge_index[0].astype(jnp.int32)
    dst = edge_index[1].astype(jnp.int32)
    deg2 = _sc_degree(dst)
    t1 = _tc1(x, W1, deg2)
    agg1 = _sc_aggregate(t1, src, dst)
    t2 = _tc2(t1, agg1, deg2, b1, W2)
    agg2 = _sc_aggregate(t2, src, dst)
    return _tc3(t2, agg2, deg2, b2)


# speedup vs baseline: 31.2105x; 2.3857x over previous
import functools

import jax
import jax.numpy as jnp
from jax import lax
from jax.experimental import pallas as pl
from jax.experimental.pallas import tpu as pltpu
from jax.experimental.pallas import tpu_sc as plsc

N = 10000
D = 128
E = 320000
NC = 2
NS = 16
NW = NC * NS
E_PER_W = E // NW
CHUNK = 100
N_CHUNKS = E_PER_W // CHUNK
NBUF = 2
RING = 4
GROUPS = N_CHUNKS // RING
WB = 40
N_WBCH = N // WB
WB_PER_TILE = -(-N_WBCH // NS)
DEG_W = 16
TC_B = 1000
TC_GRID = N // TC_B


def _mesh():
    return plsc.VectorSubcoreMesh(
        core_axis_name="c", subcore_axis_name="s", num_cores=NC, num_subcores=NS
    )


def _zero_f32(ref, rows, width):
    zeros = jnp.zeros((16,), jnp.float32)
    def body(r, _):
        for j in range(width // 16):
            ref[r, pl.ds(j * 16, 16)] = zeros
        return 0
    lax.fori_loop(0, rows, body, 0)


def _sc_degree(dstr):
    @functools.partial(
        pl.kernel,
        out_type=jax.ShapeDtypeStruct((NC * N, DEG_W), jnp.float32),
        mesh=_mesh(),
        scratch_types=[
            pltpu.VMEM((N_CHUNKS, CHUNK), jnp.int32),
            pltpu.VMEM((CHUNK, DEG_W), jnp.float32),
            pltpu.VMEM((WB, DEG_W), jnp.float32),
            pltpu.VMEM_SHARED((N, DEG_W), jnp.float32),
        ],
        compiler_params=pltpu.CompilerParams(use_tc_tiling_on_sc=False),
    )
    def deg_kernel(dstr_hbm, out_hbm, idx_v, ones_v, buf_v, acc_sh):
        c = lax.axis_index("c")
        s = lax.axis_index("s")
        wid = c * NS + s

        pltpu.sync_copy(dstr_hbm.at[wid], idx_v)
        ones = jnp.full((16,), 1.0, jnp.float32)
        def fill_ones(r, _):
            ones_v[r, pl.ds(0, 16)] = ones
            return 0
        lax.fori_loop(0, CHUNK, fill_ones, 0)
        _zero_f32(buf_v, WB, DEG_W)
        for jj in range(WB_PER_TILE):
            j = jj * NS + s
            @pl.when(j < N_WBCH)
            def _():
                pltpu.sync_copy(buf_v, acc_sh.at[pl.ds(j * WB, WB)])
        plsc.subcore_barrier()

        def body(ch, _):
            pltpu.sync_copy(ones_v, acc_sh.at[idx_v.at[ch]], add=True)
            return 0
        lax.fori_loop(0, N_CHUNKS, body, 0)
        plsc.subcore_barrier()

        for jj in range(WB_PER_TILE):
            j = jj * NS + s
            @pl.when(j < N_WBCH)
            def _():
                row = pl.multiple_of(j * WB, 8)
                pltpu.sync_copy(acc_sh.at[pl.ds(row, WB)], buf_v)
                pltpu.sync_copy(buf_v, out_hbm.at[pl.ds(c * N + row, WB)])

    return deg_kernel(dstr)


def _sc_aggregate(t, idxr):
    @functools.partial(
        pl.kernel,
        out_type=jax.ShapeDtypeStruct((NC * N, D), jnp.float32),
        mesh=_mesh(),
        scratch_types=[
            pltpu.VMEM_SHARED((N, D), jnp.float32),
            pltpu.VMEM((RING, 2, CHUNK), jnp.int32),
            pltpu.VMEM((NBUF, CHUNK, D), jnp.float32),
            [pltpu.SemaphoreType.DMA] * NBUF,
            [pltpu.SemaphoreType.DMA] * RING,
        ],
    )
    def agg_kernel(t_hbm, idxr_hbm, out_hbm, acc_sh, idx_v, rows_v,
                   gsems, isems):
        c = lax.axis_index("c")
        s = lax.axis_index("s")
        wid = c * NS + s
        idxw = idxr_hbm.at[wid]

        buf_v = rows_v.at[0].at[pl.ds(0, WB)]
        _zero_f32(buf_v, WB, D)
        for jj in range(WB_PER_TILE):
            j = jj * NS + s
            @pl.when(j < N_WBCH)
            def _():
                pltpu.sync_copy(buf_v, acc_sh.at[pl.ds(j * WB, WB)])
        plsc.subcore_barrier()

        for r in range(RING):
            pltpu.async_copy(idxw.at[r], idx_v.at[r], isems[r])
        for b in range(NBUF):
            pltpu.make_async_copy(idxw.at[0], idx_v.at[b], isems[b]).wait()
            pltpu.async_copy(t_hbm.at[idx_v.at[b, 0]], rows_v.at[b], gsems[b])

        def group(g, _):
            for b in range(RING):
                ch = g * RING + b
                buf = b % NBUF
                pltpu.make_async_copy(t_hbm.at[idx_v.at[b, 0]],
                                      rows_v.at[buf], gsems[buf]).wait()
                pltpu.sync_copy(rows_v.at[buf], acc_sh.at[idx_v.at[b, 1]],
                                add=True)
                nxt_i = ch + RING
                @pl.when(nxt_i < N_CHUNKS)
                def _():
                    pltpu.async_copy(idxw.at[nxt_i], idx_v.at[b], isems[b])
                nxt_g = ch + NBUF
                gslot = (b + NBUF) % RING
                @pl.when(nxt_g < N_CHUNKS)
                def _():
                    pltpu.make_async_copy(idxw.at[0], idx_v.at[gslot],
                                          isems[gslot]).wait()
                    pltpu.async_copy(t_hbm.at[idx_v.at[gslot, 0]],
                                     rows_v.at[buf], gsems[buf])
            return 0
        lax.fori_loop(0, GROUPS, group, 0)
        plsc.subcore_barrier()

        wb_v = rows_v.at[0].at[pl.ds(0, WB)]
        for jj in range(WB_PER_TILE):
            j = jj * NS + s
            @pl.when(j < N_WBCH)
            def _():
                row = pl.multiple_of(j * WB, 8)
                pltpu.sync_copy(acc_sh.at[pl.ds(row, WB)], wb_v)
                pltpu.sync_copy(wb_v, out_hbm.at[pl.ds(c * N + row, WB)])

    return agg_kernel(t, idxr)


def _dinv_block(d0_ref, d1_ref):
    deg = d0_ref[:, 0:1] + d1_ref[:, 0:1] + 1.0
    return lax.rsqrt(deg)


def _tc1_body(x_ref, w_ref, d0_ref, d1_ref, o_ref):
    dinv = _dinv_block(d0_ref, d1_ref)
    o_ref[...] = dinv * jnp.dot(x_ref[...], w_ref[...],
                                preferred_element_type=jnp.float32)


def _tc2_body(t_ref, a0_ref, a1_ref, d0_ref, d1_ref, b_ref, w_ref, o_ref):
    dinv = _dinv_block(d0_ref, d1_ref)
    z = dinv * (a0_ref[...] + a1_ref[...] + t_ref[...]) + b_ref[...]
    h = jnp.maximum(z, 0.0)
    o_ref[...] = dinv * jnp.dot(h, w_ref[...], preferred_element_type=jnp.float32)


def _tc3_body(t_ref, a0_ref, a1_ref, d0_ref, d1_ref, b_ref, o_ref):
    dinv = _dinv_block(d0_ref, d1_ref)
    z = dinv * (a0_ref[...] + a1_ref[...] + t_ref[...]) + b_ref[...]
    m = jnp.max(z, axis=1, keepdims=True)
    ez = jnp.exp(z - m)
    o_ref[...] = z - m - jnp.log(jnp.sum(ez, axis=1, keepdims=True))


_row_blk = pl.BlockSpec((TC_B, D), lambda i: (i, 0))
_deg0_blk = pl.BlockSpec((TC_B, DEG_W), lambda i: (i, 0))
_deg1_blk = pl.BlockSpec((TC_B, DEG_W), lambda i: (i + TC_GRID, 0))
_agg0_blk = pl.BlockSpec((TC_B, D), lambda i: (i, 0))
_agg1_blk = pl.BlockSpec((TC_B, D), lambda i: (i + TC_GRID, 0))
_w_blk = pl.BlockSpec((D, D), lambda i: (0, 0))
_b_blk = pl.BlockSpec((1, D), lambda i: (0, 0))
_out_sds = jax.ShapeDtypeStruct((N, D), jnp.float32)


def _tc1(x, W1, deg2):
    return pl.pallas_call(
        _tc1_body,
        grid=(TC_GRID,),
        in_specs=[_row_blk, _w_blk, _deg0_blk, _deg1_blk],
        out_specs=_row_blk,
        out_shape=_out_sds,
    )(x, W1, deg2, deg2)


def _tc2(t1, agg1, deg2, b1, W2):
    return pl.pallas_call(
        _tc2_body,
        grid=(TC_GRID,),
        in_specs=[_row_blk, _agg0_blk, _agg1_blk, _deg0_blk, _deg1_blk,
                  _b_blk, _w_blk],
        out_specs=_row_blk,
        out_shape=_out_sds,
    )(t1, agg1, agg1, deg2, deg2, b1.reshape(1, D), W2)


def _tc3(t2, agg2, deg2, b2):
    return pl.pallas_call(
        _tc3_body,
        grid=(TC_GRID,),
        in_specs=[_row_blk, _agg0_blk, _agg1_blk, _deg0_blk, _deg1_blk, _b_blk],
        out_specs=_row_blk,
        out_shape=_out_sds,
    )(t2, agg2, agg2, deg2, deg2, b2.reshape(1, D))


def kernel(x, edge_index, W1, b1, W2, b2):
    srcr = edge_index[0].astype(jnp.int32).reshape(NW, N_CHUNKS, CHUNK)
    dstr = edge_index[1].astype(jnp.int32).reshape(NW, N_CHUNKS, CHUNK)
    idxr = jnp.stack([srcr, dstr], axis=2)
    deg2 = _sc_degree(dstr)
    t1 = _tc1(x, W1, deg2)
    agg1 = _sc_aggregate(t1, idxr)
    t2 = _tc2(t1, agg1, deg2, b1, W2)
    agg2 = _sc_aggregate(t2, idxr)
    return _tc3(t2, agg2, deg2, b2)


# speedup vs baseline: 32.1944x; 1.0315x over previous
import functools

import jax
import jax.numpy as jnp
from jax import lax
from jax.experimental import pallas as pl
from jax.experimental.pallas import tpu as pltpu
from jax.experimental.pallas import tpu_sc as plsc

N = 10000
D = 128
E = 320000
NC = 2
NS = 16
NW = NC * NS
E_PER_W = E // NW
CHUNK = 125
N_CHUNKS = E_PER_W // CHUNK
NBUF = 2
RING = 4
GROUPS = N_CHUNKS // RING
WB = 40
N_WBCH = N // WB
WB_PER_TILE = -(-N_WBCH // NS)
DEG_W = 16
TC_B = 1000
TC_GRID = N // TC_B


def _mesh():
    return plsc.VectorSubcoreMesh(
        core_axis_name="c", subcore_axis_name="s", num_cores=NC, num_subcores=NS
    )


def _zero_f32(ref, rows, width):
    zeros = jnp.zeros((16,), jnp.float32)
    def body(r, _):
        for j in range(width // 16):
            ref[r, pl.ds(j * 16, 16)] = zeros
        return 0
    lax.fori_loop(0, rows, body, 0)


def _sc_degree(dstr):
    @functools.partial(
        pl.kernel,
        out_type=jax.ShapeDtypeStruct((NC * N, DEG_W), jnp.float32),
        mesh=_mesh(),
        scratch_types=[
            pltpu.VMEM((N_CHUNKS, CHUNK), jnp.int32),
            pltpu.VMEM((CHUNK, DEG_W), jnp.float32),
            pltpu.VMEM((WB, DEG_W), jnp.float32),
            pltpu.VMEM_SHARED((N, DEG_W), jnp.float32),
        ],
        compiler_params=pltpu.CompilerParams(use_tc_tiling_on_sc=False),
    )
    def deg_kernel(dstr_hbm, out_hbm, idx_v, ones_v, buf_v, acc_sh):
        c = lax.axis_index("c")
        s = lax.axis_index("s")
        wid = c * NS + s

        pltpu.sync_copy(dstr_hbm.at[wid], idx_v)
        ones = jnp.full((16,), 1.0, jnp.float32)
        def fill_ones(r, _):
            ones_v[r, pl.ds(0, 16)] = ones
            return 0
        lax.fori_loop(0, CHUNK, fill_ones, 0)
        _zero_f32(buf_v, WB, DEG_W)
        for jj in range(WB_PER_TILE):
            j = jj * NS + s
            @pl.when(j < N_WBCH)
            def _():
                pltpu.sync_copy(buf_v, acc_sh.at[pl.ds(j * WB, WB)])
        plsc.subcore_barrier()

        def body(ch, _):
            pltpu.sync_copy(ones_v, acc_sh.at[idx_v.at[ch]], add=True)
            return 0
        lax.fori_loop(0, N_CHUNKS, body, 0)
        plsc.subcore_barrier()

        for jj in range(WB_PER_TILE):
            j = jj * NS + s
            @pl.when(j < N_WBCH)
            def _():
                row = pl.multiple_of(j * WB, 8)
                pltpu.sync_copy(acc_sh.at[pl.ds(row, WB)], buf_v)
                pltpu.sync_copy(buf_v, out_hbm.at[pl.ds(c * N + row, WB)])

    return deg_kernel(dstr)


def _sc_aggregate(t, idxr):
    @functools.partial(
        pl.kernel,
        out_type=jax.ShapeDtypeStruct((NC * N, D), jnp.float32),
        mesh=_mesh(),
        scratch_types=[
            pltpu.VMEM_SHARED((N, D), jnp.float32),
            pltpu.VMEM((RING, 2, CHUNK), jnp.int32),
            pltpu.VMEM((NBUF, CHUNK, D), jnp.float32),
            [pltpu.SemaphoreType.DMA] * NBUF,
            [pltpu.SemaphoreType.DMA] * RING,
        ],
    )
    def agg_kernel(t_hbm, idxr_hbm, out_hbm, acc_sh, idx_v, rows_v,
                   gsems, isems):
        c = lax.axis_index("c")
        s = lax.axis_index("s")
        wid = c * NS + s
        idxw = idxr_hbm.at[wid]

        buf_v = rows_v.at[0].at[pl.ds(0, WB)]
        _zero_f32(buf_v, WB, D)
        for jj in range(WB_PER_TILE):
            j = jj * NS + s
            @pl.when(j < N_WBCH)
            def _():
                pltpu.sync_copy(buf_v, acc_sh.at[pl.ds(j * WB, WB)])
        plsc.subcore_barrier()

        for r in range(RING):
            pltpu.async_copy(idxw.at[r], idx_v.at[r], isems[r])
        for b in range(NBUF):
            pltpu.make_async_copy(idxw.at[0], idx_v.at[b], isems[b]).wait()
            pltpu.async_copy(t_hbm.at[idx_v.at[b, 0]], rows_v.at[b], gsems[b])

        def group(g, _):
            for b in range(RING):
                ch = g * RING + b
                buf = b % NBUF
                pltpu.make_async_copy(t_hbm.at[idx_v.at[b, 0]],
                                      rows_v.at[buf], gsems[buf]).wait()
                pltpu.sync_copy(rows_v.at[buf], acc_sh.at[idx_v.at[b, 1]],
                                add=True)
                nxt_i = ch + RING
                @pl.when(nxt_i < N_CHUNKS)
                def _():
                    pltpu.async_copy(idxw.at[nxt_i], idx_v.at[b], isems[b])
                nxt_g = ch + NBUF
                gslot = (b + NBUF) % RING
                @pl.when(nxt_g < N_CHUNKS)
                def _():
                    pltpu.make_async_copy(idxw.at[0], idx_v.at[gslot],
                                          isems[gslot]).wait()
                    pltpu.async_copy(t_hbm.at[idx_v.at[gslot, 0]],
                                     rows_v.at[buf], gsems[buf])
            return 0
        lax.fori_loop(0, GROUPS, group, 0)
        plsc.subcore_barrier()

        wb_v = rows_v.at[0].at[pl.ds(0, WB)]
        for jj in range(WB_PER_TILE):
            j = jj * NS + s
            @pl.when(j < N_WBCH)
            def _():
                row = pl.multiple_of(j * WB, 8)
                pltpu.sync_copy(acc_sh.at[pl.ds(row, WB)], wb_v)
                pltpu.sync_copy(wb_v, out_hbm.at[pl.ds(c * N + row, WB)])

    return agg_kernel(t, idxr)


def _dinv_block(d0_ref, d1_ref):
    deg = d0_ref[:, 0:1] + d1_ref[:, 0:1] + 1.0
    return lax.rsqrt(deg)


def _tc1_body(x_ref, w_ref, d0_ref, d1_ref, o_ref):
    dinv = _dinv_block(d0_ref, d1_ref)
    o_ref[...] = dinv * jnp.dot(x_ref[...], w_ref[...],
                                preferred_element_type=jnp.float32)


def _tc2_body(t_ref, a0_ref, a1_ref, d0_ref, d1_ref, b_ref, w_ref, o_ref):
    dinv = _dinv_block(d0_ref, d1_ref)
    z = dinv * (a0_ref[...] + a1_ref[...] + t_ref[...]) + b_ref[...]
    h = jnp.maximum(z, 0.0)
    o_ref[...] = dinv * jnp.dot(h, w_ref[...], preferred_element_type=jnp.float32)


def _tc3_body(t_ref, a0_ref, a1_ref, d0_ref, d1_ref, b_ref, o_ref):
    dinv = _dinv_block(d0_ref, d1_ref)
    z = dinv * (a0_ref[...] + a1_ref[...] + t_ref[...]) + b_ref[...]
    m = jnp.max(z, axis=1, keepdims=True)
    ez = jnp.exp(z - m)
    o_ref[...] = z - m - jnp.log(jnp.sum(ez, axis=1, keepdims=True))


_row_blk = pl.BlockSpec((TC_B, D), lambda i: (i, 0))
_deg0_blk = pl.BlockSpec((TC_B, DEG_W), lambda i: (i, 0))
_deg1_blk = pl.BlockSpec((TC_B, DEG_W), lambda i: (i + TC_GRID, 0))
_agg0_blk = pl.BlockSpec((TC_B, D), lambda i: (i, 0))
_agg1_blk = pl.BlockSpec((TC_B, D), lambda i: (i + TC_GRID, 0))
_w_blk = pl.BlockSpec((D, D), lambda i: (0, 0))
_b_blk = pl.BlockSpec((1, D), lambda i: (0, 0))
_out_sds = jax.ShapeDtypeStruct((N, D), jnp.float32)


def _tc1(x, W1, deg2):
    return pl.pallas_call(
        _tc1_body,
        grid=(TC_GRID,),
        in_specs=[_row_blk, _w_blk, _deg0_blk, _deg1_blk],
        out_specs=_row_blk,
        out_shape=_out_sds,
    )(x, W1, deg2, deg2)


def _tc2(t1, agg1, deg2, b1, W2):
    return pl.pallas_call(
        _tc2_body,
        grid=(TC_GRID,),
        in_specs=[_row_blk, _agg0_blk, _agg1_blk, _deg0_blk, _deg1_blk,
                  _b_blk, _w_blk],
        out_specs=_row_blk,
        out_shape=_out_sds,
    )(t1, agg1, agg1, deg2, deg2, b1.reshape(1, D), W2)


def _tc3(t2, agg2, deg2, b2):
    return pl.pallas_call(
        _tc3_body,
        grid=(TC_GRID,),
        in_specs=[_row_blk, _agg0_blk, _agg1_blk, _deg0_blk, _deg1_blk, _b_blk],
        out_specs=_row_blk,
        out_shape=_out_sds,
    )(t2, agg2, agg2, deg2, deg2, b2.reshape(1, D))


def kernel(x, edge_index, W1, b1, W2, b2):
    srcr = edge_index[0].astype(jnp.int32).reshape(NW, N_CHUNKS, CHUNK)
    dstr = edge_index[1].astype(jnp.int32).reshape(NW, N_CHUNKS, CHUNK)
    idxr = jnp.stack([srcr, dstr], axis=2)
    deg2 = _sc_degree(dstr)
    t1 = _tc1(x, W1, deg2)
    agg1 = _sc_aggregate(t1, idxr)
    t2 = _tc2(t1, agg1, deg2, b1, W2)
    agg2 = _sc_aggregate(t2, idxr)
    return _tc3(t2, agg2, deg2, b2)
